# baseline (speedup 1.0000x reference)
import functools

import jax
import jax.numpy as jnp
from jax import lax
from jax.experimental import pallas as pl
from jax.experimental.pallas import tpu as pltpu
from jax.experimental.pallas import tpu_sc as plsc

N_NODES = 50000
B = 50000
K = 10
D = 128
E = 128
NW = 32
GN = 16
NSTREAM = 2
S_STEPS = 98
B_PAD = NW * S_STEPS * GN
NB = 3

RPS = GN * K // NSTREAM
GPS = GN // NSTREAM


def _prep_idx(pos_neigh, neg_neigh):
    def one(neigh):
        flat = neigh.astype(jnp.int32).reshape(-1)
        flat = jnp.pad(flat, (0, B_PAD * K - B * K))
        return flat.reshape(NW, S_STEPS, 1, NSTREAM, RPS)
    return jnp.concatenate([one(pos_neigh), one(neg_neigh)], axis=2)


def _sc_gather_sum(features, idx):
    info = plsc.get_sparse_core_info()
    nc = info.num_cores

    mesh = plsc.VectorSubcoreMesh(core_axis_name="c", subcore_axis_name="s")

    @functools.partial(
        pl.kernel,
        out_type=(jax.ShapeDtypeStruct((B_PAD, D), jnp.float32),
                  jax.ShapeDtypeStruct((B_PAD, D), jnp.float32)),
        mesh=mesh,
        scratch_types=[
            pltpu.VMEM((S_STEPS, 2, NSTREAM, RPS), jnp.int32),
            pltpu.VMEM((NB, GN * K, D), jnp.float32),
            pltpu.VMEM((2, 2, GN, D), jnp.float32),
            pltpu.SemaphoreType.DMA,
            pltpu.SemaphoreType.DMA,
        ],
    )
    def k(feat_hbm, idx_hbm, out_p_hbm, out_n_hbm,
          idx_v, rows_v, acc_v, sem_g, sem_o):
        wid = lax.axis_index("s") * nc + lax.axis_index("c")
        pltpu.sync_copy(idx_hbm.at[wid], idx_v)

        nsteps = 2 * S_STEPS

        def gather_args(t, h):
            s, pn, slot = t // 2, t % 2, t % NB
            return (feat_hbm.at[idx_v.at[s, pn, h]],
                    rows_v.at[slot, pl.ds(h * RPS, RPS)], sem_g)

        def issue(t):
            for h in range(NSTREAM):
                pltpu.async_copy(*gather_args(t, h))

        def out_args(s):
            par = s % 2
            base = wid * (S_STEPS * GN) + s * GN
            return [(acc_v.at[par, 0], out_p_hbm.at[pl.ds(base, GN)], sem_o),
                    (acc_v.at[par, 1], out_n_hbm.at[pl.ds(base, GN)], sem_o)]

        issue(0)
        issue(1)

        def body(t, _):
            s, pn, slot = t // 2, t % 2, t % NB
            par = s % 2

            @pl.when(t + 2 < nsteps)
            def _():
                issue(t + 2)

            @pl.when((pn == 0) & (s >= 2))
            def _():
                for a in out_args(s - 2):
                    pltpu.make_async_copy(*a).wait()

            for h in range(NSTREAM):
                pltpu.make_async_copy(*gather_args(t, h)).wait()

                def grp(g, _):
                    for c in range(D // 16):
                        sl = pl.ds(c * 16, 16)
                        a = rows_v[slot, g * K + 0, sl]
                        for j in range(1, K):
                            a = a + rows_v[slot, g * K + j, sl]
                        acc_v[par, pn, g, sl] = a
                    return 0

                lax.fori_loop(h * GPS, (h + 1) * GPS, grp, 0, unroll=False)

            @pl.when(pn == 1)
            def _():
                for a in out_args(s):
                    pltpu.async_copy(*a)

            return 0

        lax.fori_loop(0, nsteps, body, 0, unroll=False)

        for s in (S_STEPS - 2, S_STEPS - 1):
            for a in out_args(s):
                pltpu.make_async_copy(*a).wait()

    return k(features, idx)


def _tc_project(s_pos, s_neg, w_bal, w_unbal):
    blk = 512
    grid = (pl.cdiv(B, blk),)
    dn = (((1,), (1,)), ((), ()))

    def body(sp_ref, sn_ref, wb_ref, wu_ref, ob_ref, ou_ref):
        scale = jnp.float32(1.0 / K)
        ob_ref[...] = jnp.tanh(scale * lax.dot_general(
            wb_ref[...], sp_ref[...], dn, preferred_element_type=jnp.float32))
        ou_ref[...] = jnp.tanh(scale * lax.dot_general(
            wu_ref[...], sn_ref[...], dn, preferred_element_type=jnp.float32))

    return pl.pallas_call(
        body,
        grid=grid,
        in_specs=[
            pl.BlockSpec((blk, D), lambda i: (i, 0)),
            pl.BlockSpec((blk, D), lambda i: (i, 0)),
            pl.BlockSpec((E, D), lambda i: (0, 0)),
            pl.BlockSpec((E, D), lambda i: (0, 0)),
        ],
        out_specs=[
            pl.BlockSpec((E, blk), lambda i: (0, i)),
            pl.BlockSpec((E, blk), lambda i: (0, i)),
        ],
        out_shape=[
            jax.ShapeDtypeStruct((E, B), jnp.float32),
            jax.ShapeDtypeStruct((E, B), jnp.float32),
        ],
    )(s_pos, s_neg, w_bal, w_unbal)


def kernel(nodes, pos_neigh, neg_neigh, features, W_bal, W_unbal):
    idx = _prep_idx(pos_neigh, neg_neigh)
    s_pos, s_neg = _sc_gather_sum(features, idx)
    mapped_bal, mapped_unbal = _tc_project(s_pos, s_neg, W_bal, W_unbal)
    return (mapped_bal, mapped_unbal)


# speedup vs baseline: 1.0339x; 1.0339x over previous
import functools

import jax
import jax.numpy as jnp
from jax import lax
from jax.experimental import pallas as pl
from jax.experimental.pallas import tpu as pltpu
from jax.experimental.pallas import tpu_sc as plsc

N_NODES = 50000
B = 50000
K = 10
D = 128
E = 128
NW = 32
GN = 16
NSTREAM = 2
S_STEPS = 98
B_PAD = NW * S_STEPS * GN
NB = 3

RPS = GN * K // NSTREAM
GPS = GN // NSTREAM
IPW = S_STEPS * GN * K
IREAL = B * K


def _sc_gather_sum(features, pos_flat, neg_flat):
    info = plsc.get_sparse_core_info()
    nc = info.num_cores

    mesh = plsc.VectorSubcoreMesh(core_axis_name="c", subcore_axis_name="s")

    @functools.partial(
        pl.kernel,
        out_type=(jax.ShapeDtypeStruct((B_PAD, D), jnp.float32),
                  jax.ShapeDtypeStruct((B_PAD, D), jnp.float32)),
        mesh=mesh,
        scratch_types=[
            pltpu.VMEM((2 * IPW,), jnp.int32),
            pltpu.VMEM((NB, GN * K, D), jnp.float32),
            pltpu.VMEM((2, 2, GN, D), jnp.float32),
            pltpu.SemaphoreType.DMA,
            pltpu.SemaphoreType.DMA,
        ],
    )
    def k(feat_hbm, pos_hbm, neg_hbm, out_p_hbm, out_n_hbm,
          idx_v, rows_v, acc_v, sem_g, sem_o):
        wid = lax.axis_index("s") * nc + lax.axis_index("c")

        ibase = wid * IPW
        nvalid = IREAL - (NW - 1) * IPW

        @pl.when(wid < NW - 1)
        def _():
            pltpu.sync_copy(pos_hbm.at[pl.ds(ibase, IPW)],
                            idx_v.at[pl.ds(0, IPW)])
            pltpu.sync_copy(neg_hbm.at[pl.ds(ibase, IPW)],
                            idx_v.at[pl.ds(IPW, IPW)])

        @pl.when(wid == NW - 1)
        def _():
            pltpu.sync_copy(pos_hbm.at[pl.ds(ibase, nvalid)],
                            idx_v.at[pl.ds(0, nvalid)])
            pltpu.sync_copy(neg_hbm.at[pl.ds(ibase, nvalid)],
                            idx_v.at[pl.ds(IPW, nvalid)])
            zero16 = jnp.zeros((16,), jnp.int32)

            def zfill(i, _):
                for pn in range(2):
                    idx_v[pl.ds(pn * IPW + nvalid + i * 16, 16)] = zero16
                return 0

            lax.fori_loop(0, (IPW - nvalid) // 16, zfill, 0, unroll=False)

        nsteps = 2 * S_STEPS

        def gather_args(t, h):
            s, pn, slot = t // 2, t % 2, t % NB
            return (feat_hbm.at[idx_v.at[pl.ds(pn * IPW + s * (GN * K) + h * RPS,
                                                RPS)]],
                    rows_v.at[slot, pl.ds(h * RPS, RPS)], sem_g)

        def issue(t):
            for h in range(NSTREAM):
                pltpu.async_copy(*gather_args(t, h))

        def out_args(s):
            par = s % 2
            base = wid * (S_STEPS * GN) + s * GN
            return [(acc_v.at[par, 0], out_p_hbm.at[pl.ds(base, GN)], sem_o),
                    (acc_v.at[par, 1], out_n_hbm.at[pl.ds(base, GN)], sem_o)]

        issue(0)
        issue(1)

        def body(t, _):
            s, pn, slot = t // 2, t % 2, t % NB
            par = s % 2

            @pl.when(t + 2 < nsteps)
            def _():
                issue(t + 2)

            @pl.when((pn == 0) & (s >= 2))
            def _():
                for a in out_args(s - 2):
                    pltpu.make_async_copy(*a).wait()

            for h in range(NSTREAM):
                pltpu.make_async_copy(*gather_args(t, h)).wait()

                def grp(g, _):
                    for c in range(D // 16):
                        sl = pl.ds(c * 16, 16)
                        a = rows_v[slot, g * K + 0, sl]
                        for j in range(1, K):
                            a = a + rows_v[slot, g * K + j, sl]
                        acc_v[par, pn, g, sl] = a
                    return 0

                lax.fori_loop(h * GPS, (h + 1) * GPS, grp, 0, unroll=False)

            @pl.when(pn == 1)
            def _():
                for a in out_args(s):
                    pltpu.async_copy(*a)

            return 0

        lax.fori_loop(0, nsteps, body, 0, unroll=False)

        for s in (S_STEPS - 2, S_STEPS - 1):
            for a in out_args(s):
                pltpu.make_async_copy(*a).wait()

    return k(features, pos_flat, neg_flat)


def _tc_project(s_pos, s_neg, w_bal, w_unbal):
    blk = 512
    grid = (pl.cdiv(B, blk),)
    dn = (((1,), (1,)), ((), ()))

    def body(sp_ref, sn_ref, wb_ref, wu_ref, ob_ref, ou_ref):
        scale = jnp.float32(1.0 / K)
        ob_ref[...] = jnp.tanh(scale * lax.dot_general(
            wb_ref[...], sp_ref[...], dn, preferred_element_type=jnp.float32))
        ou_ref[...] = jnp.tanh(scale * lax.dot_general(
            wu_ref[...], sn_ref[...], dn, preferred_element_type=jnp.float32))

    return pl.pallas_call(
        body,
        grid=grid,
        in_specs=[
            pl.BlockSpec((blk, D), lambda i: (i, 0)),
            pl.BlockSpec((blk, D), lambda i: (i, 0)),
            pl.BlockSpec((E, D), lambda i: (0, 0)),
            pl.BlockSpec((E, D), lambda i: (0, 0)),
        ],
        out_specs=[
            pl.BlockSpec((E, blk), lambda i: (0, i)),
            pl.BlockSpec((E, blk), lambda i: (0, i)),
        ],
        out_shape=[
            jax.ShapeDtypeStruct((E, B), jnp.float32),
            jax.ShapeDtypeStruct((E, B), jnp.float32),
        ],
    )(s_pos, s_neg, w_bal, w_unbal)


def kernel(nodes, pos_neigh, neg_neigh, features, W_bal, W_unbal):
    pos_flat = pos_neigh.astype(jnp.int32).reshape(-1)
    neg_flat = neg_neigh.astype(jnp.int32).reshape(-1)
    s_pos, s_neg = _sc_gather_sum(features, pos_flat, neg_flat)
    mapped_bal, mapped_unbal = _tc_project(s_pos, s_neg, W_bal, W_unbal)
    return (mapped_bal, mapped_unbal)


# speedup vs baseline: 1.0959x; 1.0600x over previous
import functools

import jax
import jax.numpy as jnp
from jax import lax
from jax.experimental import pallas as pl
from jax.experimental.pallas import tpu as pltpu
from jax.experimental.pallas import tpu_sc as plsc

N_NODES = 50000
B = 50000
K = 10
D = 128
E = 128
NW = 32
GN = 16
NSTREAM = 2
S0 = 114
S1 = 82
SP = S0 + S1
B_PAD = 16 * SP * GN
NB = 3

RPS = GN * K // NSTREAM
GPS = GN // NSTREAM
IMAX = max(S0, S1) * GN * K
IREAL = B * K


def _sc_gather_sum(features, pos_flat, neg_flat):
    info = plsc.get_sparse_core_info()
    nc = info.num_cores

    mesh = plsc.VectorSubcoreMesh(core_axis_name="c", subcore_axis_name="s")

    @functools.partial(
        pl.kernel,
        out_type=(jax.ShapeDtypeStruct((B_PAD, D), jnp.float32),
                  jax.ShapeDtypeStruct((B_PAD, D), jnp.float32)),
        mesh=mesh,
        scratch_types=[
            pltpu.VMEM((2 * IMAX,), jnp.int32),
            pltpu.VMEM((NB, GN * K, D), jnp.float32),
            pltpu.VMEM((2, 2, GN, D), jnp.float32),
            pltpu.SemaphoreType.DMA,
            pltpu.SemaphoreType.DMA,
        ],
    )
    def k(feat_hbm, pos_hbm, neg_hbm, out_p_hbm, out_n_hbm,
          idx_v, rows_v, acc_v, sem_g, sem_o):
        sid = lax.axis_index("s")
        cid = lax.axis_index("c")

        def run(s_steps, node_base, tail_worker):
            ipw = s_steps * GN * K
            ibase = node_base * K

            def stage(pn, src_hbm):
                if tail_worker is None:
                    pltpu.sync_copy(src_hbm.at[pl.ds(ibase, ipw)],
                                    idx_v.at[pl.ds(pn * IMAX, ipw)])
                else:
                    nvalid, nfill = tail_worker

                    @pl.when(sid < 15)
                    def _():
                        pltpu.sync_copy(src_hbm.at[pl.ds(ibase, ipw)],
                                        idx_v.at[pl.ds(pn * IMAX, ipw)])

                    @pl.when(sid == 15)
                    def _():
                        pltpu.sync_copy(src_hbm.at[pl.ds(ibase, nvalid)],
                                        idx_v.at[pl.ds(pn * IMAX, nvalid)])
                        zero16 = jnp.zeros((16,), jnp.int32)

                        def zfill(i, _):
                            idx_v[pl.ds(pn * IMAX + nvalid + i * 16, 16)] = zero16
                            return 0

                        lax.fori_loop(0, nfill // 16, zfill, 0, unroll=False)

            stage(0, pos_hbm)
            stage(1, neg_hbm)

            nsteps = 2 * s_steps

            def gather_args(t, h):
                s, pn, slot = t // 2, t % 2, t % NB
                off = pn * IMAX + s * (GN * K) + h * RPS
                return (feat_hbm.at[idx_v.at[pl.ds(off, RPS)]],
                        rows_v.at[slot, pl.ds(h * RPS, RPS)], sem_g)

            def issue(t):
                for h in range(NSTREAM):
                    pltpu.async_copy(*gather_args(t, h))

            def out_args(s):
                par = s % 2
                base = node_base + s * GN
                return [(acc_v.at[par, 0], out_p_hbm.at[pl.ds(base, GN)], sem_o),
                        (acc_v.at[par, 1], out_n_hbm.at[pl.ds(base, GN)], sem_o)]

            issue(0)
            issue(1)

            def body(t, _):
                s, pn, slot = t // 2, t % 2, t % NB
                par = s % 2

                @pl.when(t + 2 < nsteps)
                def _():
                    issue(t + 2)

                @pl.when((pn == 0) & (s >= 2))
                def _():
                    for a in out_args(s - 2):
                        pltpu.make_async_copy(*a).wait()

                for h in range(NSTREAM):
                    pltpu.make_async_copy(*gather_args(t, h)).wait()

                    def grp(g, _):
                        for c in range(D // 16):
                            sl = pl.ds(c * 16, 16)
                            a = rows_v[slot, g * K + 0, sl]
                            for j in range(1, K):
                                a = a + rows_v[slot, g * K + j, sl]
                            acc_v[par, pn, g, sl] = a
                        return 0

                    lax.fori_loop(h * GPS, (h + 1) * GPS, grp, 0, unroll=False)

                @pl.when(pn == 1)
                def _():
                    for a in out_args(s):
                        pltpu.async_copy(*a)

                return 0

            lax.fori_loop(0, nsteps, body, 0, unroll=False)

            for s in (s_steps - 2, s_steps - 1):
                for a in out_args(s):
                    pltpu.make_async_copy(*a).wait()

        last_base = (15 * SP + S0) * GN
        nvalid = IREAL - last_base * K
        nfill = S1 * GN * K - nvalid

        @pl.when(cid == 0)
        def _():
            run(S0, sid * (SP * GN), None)

        @pl.when(cid == 1)
        def _():
            run(S1, sid * (SP * GN) + S0 * GN, (nvalid, nfill))

    return k(features, pos_flat, neg_flat)


def _tc_project(s_pos, s_neg, w_bal, w_unbal):
    blk = 512
    grid = (pl.cdiv(B, blk),)
    dn = (((1,), (1,)), ((), ()))

    def body(sp_ref, sn_ref, wb_ref, wu_ref, ob_ref, ou_ref):
        scale = jnp.float32(1.0 / K)
        ob_ref[...] = jnp.tanh(scale * lax.dot_general(
            wb_ref[...], sp_ref[...], dn, preferred_element_type=jnp.float32))
        ou_ref[...] = jnp.tanh(scale * lax.dot_general(
            wu_ref[...], sn_ref[...], dn, preferred_element_type=jnp.float32))

    return pl.pallas_call(
        body,
        grid=grid,
        in_specs=[
            pl.BlockSpec((blk, D), lambda i: (i, 0)),
            pl.BlockSpec((blk, D), lambda i: (i, 0)),
            pl.BlockSpec((E, D), lambda i: (0, 0)),
            pl.BlockSpec((E, D), lambda i: (0, 0)),
        ],
        out_specs=[
            pl.BlockSpec((E, blk), lambda i: (0, i)),
            pl.BlockSpec((E, blk), lambda i: (0, i)),
        ],
        out_shape=[
            jax.ShapeDtypeStruct((E, B), jnp.float32),
            jax.ShapeDtypeStruct((E, B), jnp.float32),
        ],
    )(s_pos, s_neg, w_bal, w_unbal)


def kernel(nodes, pos_neigh, neg_neigh, features, W_bal, W_unbal):
    pos_flat = pos_neigh.astype(jnp.int32).reshape(-1)
    neg_flat = neg_neigh.astype(jnp.int32).reshape(-1)
    s_pos, s_neg = _sc_gather_sum(features, pos_flat, neg_flat)
    mapped_bal, mapped_unbal = _tc_project(s_pos, s_neg, W_bal, W_unbal)
    return (mapped_bal, mapped_unbal)


# speedup vs baseline: 1.1166x; 1.0189x over previous
import functools

import jax
import jax.numpy as jnp
from jax import lax
from jax.experimental import pallas as pl
from jax.experimental.pallas import tpu as pltpu
from jax.experimental.pallas import tpu_sc as plsc

N_NODES = 50000
B = 50000
K = 10
D = 128
E = 128
NW = 32
GN = 16
NSTREAM = 2
S0 = 119
S1 = 77
SP = S0 + S1
B_PAD = 16 * SP * GN
NB = 3

RPS = GN * K // NSTREAM
GPS = GN // NSTREAM
IMAX = max(S0, S1) * GN * K
IREAL = B * K


def _sc_gather_sum(features, pos_flat, neg_flat):
    info = plsc.get_sparse_core_info()
    nc = info.num_cores

    mesh = plsc.VectorSubcoreMesh(core_axis_name="c", subcore_axis_name="s")

    @functools.partial(
        pl.kernel,
        out_type=(jax.ShapeDtypeStruct((B_PAD, D), jnp.float32),
                  jax.ShapeDtypeStruct((B_PAD, D), jnp.float32)),
        mesh=mesh,
        scratch_types=[
            pltpu.VMEM((2 * IMAX,), jnp.int32),
            pltpu.VMEM((NB, GN * K, D), jnp.float32),
            pltpu.VMEM((2, 2, GN, D), jnp.float32),
            pltpu.SemaphoreType.DMA,
            pltpu.SemaphoreType.DMA,
        ],
    )
    def k(feat_hbm, pos_hbm, neg_hbm, out_p_hbm, out_n_hbm,
          idx_v, rows_v, acc_v, sem_g, sem_o):
        sid = lax.axis_index("s")
        cid = lax.axis_index("c")

        def run(s_steps, node_base, tail_worker):
            ipw = s_steps * GN * K
            ibase = node_base * K

            def stage(pn, src_hbm):
                if tail_worker is None:
                    pltpu.sync_copy(src_hbm.at[pl.ds(ibase, ipw)],
                                    idx_v.at[pl.ds(pn * IMAX, ipw)])
                else:
                    nvalid, nfill = tail_worker

                    @pl.when(sid < 15)
                    def _():
                        pltpu.sync_copy(src_hbm.at[pl.ds(ibase, ipw)],
                                        idx_v.at[pl.ds(pn * IMAX, ipw)])

                    @pl.when(sid == 15)
                    def _():
                        pltpu.sync_copy(src_hbm.at[pl.ds(ibase, nvalid)],
                                        idx_v.at[pl.ds(pn * IMAX, nvalid)])
                        zero16 = jnp.zeros((16,), jnp.int32)

                        def zfill(i, _):
                            idx_v[pl.ds(pn * IMAX + nvalid + i * 16, 16)] = zero16
                            return 0

                        lax.fori_loop(0, nfill // 16, zfill, 0, unroll=False)

            stage(0, pos_hbm)
            stage(1, neg_hbm)

            nsteps = 2 * s_steps

            def gather_args(t, h):
                s, pn, slot = t // 2, t % 2, t % NB
                off = pn * IMAX + s * (GN * K) + h * RPS
                return (feat_hbm.at[idx_v.at[pl.ds(off, RPS)]],
                        rows_v.at[slot, pl.ds(h * RPS, RPS)], sem_g)

            def issue(t):
                for h in range(NSTREAM):
                    pltpu.async_copy(*gather_args(t, h))

            def out_args(s):
                par = s % 2
                base = node_base + s * GN
                return [(acc_v.at[par, 0], out_p_hbm.at[pl.ds(base, GN)], sem_o),
                        (acc_v.at[par, 1], out_n_hbm.at[pl.ds(base, GN)], sem_o)]

            issue(0)
            issue(1)

            def body(t, _):
                s, pn, slot = t // 2, t % 2, t % NB
                par = s % 2

                @pl.when(t + 2 < nsteps)
                def _():
                    issue(t + 2)

                @pl.when((pn == 0) & (s >= 2))
                def _():
                    for a in out_args(s - 2):
                        pltpu.make_async_copy(*a).wait()

                for h in range(NSTREAM):
                    pltpu.make_async_copy(*gather_args(t, h)).wait()

                    def grp(g, _):
                        for c in range(D // 16):
                            sl = pl.ds(c * 16, 16)
                            a = rows_v[slot, g * K + 0, sl]
                            for j in range(1, K):
                                a = a + rows_v[slot, g * K + j, sl]
                            acc_v[par, pn, g, sl] = a
                        return 0

                    lax.fori_loop(h * GPS, (h + 1) * GPS, grp, 0, unroll=False)

                @pl.when(pn == 1)
                def _():
                    for a in out_args(s):
                        pltpu.async_copy(*a)

                return 0

            lax.fori_loop(0, nsteps, body, 0, unroll=False)

            for s in (s_steps - 2, s_steps - 1):
                for a in out_args(s):
                    pltpu.make_async_copy(*a).wait()

        last_base = (15 * SP + S0) * GN
        nvalid = IREAL - last_base * K
        nfill = S1 * GN * K - nvalid

        @pl.when(cid == 0)
        def _():
            run(S0, sid * (SP * GN), None)

        @pl.when(cid == 1)
        def _():
            run(S1, sid * (SP * GN) + S0 * GN, (nvalid, nfill))

    return k(features, pos_flat, neg_flat)


def _tc_project(s_pos, s_neg, w_bal, w_unbal):
    blk = 512
    grid = (pl.cdiv(B, blk),)
    dn = (((1,), (1,)), ((), ()))

    def body(sp_ref, sn_ref, wb_ref, wu_ref, ob_ref, ou_ref):
        scale = jnp.float32(1.0 / K)
        ob_ref[...] = jnp.tanh(scale * lax.dot_general(
            wb_ref[...], sp_ref[...], dn, preferred_element_type=jnp.float32))
        ou_ref[...] = jnp.tanh(scale * lax.dot_general(
            wu_ref[...], sn_ref[...], dn, preferred_element_type=jnp.float32))

    return pl.pallas_call(
        body,
        grid=grid,
        in_specs=[
            pl.BlockSpec((blk, D), lambda i: (i, 0)),
            pl.BlockSpec((blk, D), lambda i: (i, 0)),
            pl.BlockSpec((E, D), lambda i: (0, 0)),
            pl.BlockSpec((E, D), lambda i: (0, 0)),
        ],
        out_specs=[
            pl.BlockSpec((E, blk), lambda i: (0, i)),
            pl.BlockSpec((E, blk), lambda i: (0, i)),
        ],
        out_shape=[
            jax.ShapeDtypeStruct((E, B), jnp.float32),
            jax.ShapeDtypeStruct((E, B), jnp.float32),
        ],
    )(s_pos, s_neg, w_bal, w_unbal)


def kernel(nodes, pos_neigh, neg_neigh, features, W_bal, W_unbal):
    pos_flat = pos_neigh.astype(jnp.int32).reshape(-1)
    neg_flat = neg_neigh.astype(jnp.int32).reshape(-1)
    s_pos, s_neg = _sc_gather_sum(features, pos_flat, neg_flat)
    mapped_bal, mapped_unbal = _tc_project(s_pos, s_neg, W_bal, W_unbal)
    return (mapped_bal, mapped_unbal)
